# Initial kernel scaffold; baseline (speedup 1.0000x reference)
import jax
import jax.numpy as jnp
from jax.experimental import pallas as pl


def kernel(feature, semantic, edge_index, W1s, b1s, W1m, b1m, W2, b2, alpha, beta, num_iterations):
    raise NotImplementedError("write your pallas kernel here")


# speedup vs baseline: 4.9360x; 4.9360x over previous
import functools

import jax
import jax.numpy as jnp
from jax import lax
from jax.experimental import pallas as pl
from jax.experimental.pallas import tpu as pltpu
from jax.experimental.pallas import tpu_sc as plsc

_NC = 2
_NS = 16
_SLOPE = 0.01


def _lrelu(x):
    return jnp.where(x >= 0, x, _SLOPE * x)


def _pick_chunk(epw):
    for k in range(128, 7, -8):
        if epw % k == 0:
            return k
    raise ValueError(f"no valid chunk size for {epw} edges per worker")


def _tc_init(feature, semantic, w1s, b1s2, w1m, b1m2, alpha2, beta2, blk):
    n, f = feature.shape
    s_dim = semantic.shape[1]
    h = w1s.shape[0]

    def body(f_r, se_r, w1s_r, b1s_r, w1m_r, b1m_r, a_r, b_r, o_r):
        a = a_r[0, 0]
        b = b_r[0, 0]
        x = lax.dot_general(f_r[...], w1s_r[...], (((1,), (1,)), ((), ())),
                            preferred_element_type=jnp.float32) + b1s_r[...]
        y = lax.dot_general(se_r[...], w1m_r[...], (((1,), (1,)), ((), ())),
                            preferred_element_type=jnp.float32) + b1m_r[...]
        o_r[...] = a * _lrelu(x) + b * _lrelu(y)

    return pl.pallas_call(
        body,
        grid=(n // blk,),
        in_specs=[
            pl.BlockSpec((blk, f), lambda i: (i, 0)),
            pl.BlockSpec((blk, s_dim), lambda i: (i, 0)),
            pl.BlockSpec((h, f), lambda i: (0, 0)),
            pl.BlockSpec((1, h), lambda i: (0, 0)),
            pl.BlockSpec((h, s_dim), lambda i: (0, 0)),
            pl.BlockSpec((1, h), lambda i: (0, 0)),
            pl.BlockSpec((1, 1), lambda i: (0, 0)),
            pl.BlockSpec((1, 1), lambda i: (0, 0)),
        ],
        out_specs=pl.BlockSpec((blk, h), lambda i: (i, 0)),
        out_shape=jax.ShapeDtypeStruct((n, h), jnp.float32),
    )(feature, semantic, w1s, b1s2, w1m, b1m2, alpha2, beta2)


def _tc_dinv(degparts, npad, h):
    def body(d_r, o_r):
        dv = d_r[...]
        o_r[...] = 1.0 / jnp.maximum(dv[0] + dv[1], 1.0)

    return pl.pallas_call(
        body,
        out_shape=jax.ShapeDtypeStruct((npad, h), jnp.float32),
    )(degparts)


def _tc_update(emb, parts, dinv, w2, b22, alpha2, beta2, blk):
    n, h = emb.shape

    def body(e_r, p_r, d_r, w2_r, b2_r, a_r, b_r, o_r):
        a = a_r[0, 0]
        b = b_r[0, 0]
        pv = p_r[...]
        msum = pv[0] + pv[1]
        comb = a * e_r[...] + b * (msum * d_r[...])
        z = lax.dot_general(comb, w2_r[...], (((1,), (1,)), ((), ())),
                            preferred_element_type=jnp.float32) + b2_r[...]
        o_r[...] = _lrelu(z)

    return pl.pallas_call(
        body,
        grid=(n // blk,),
        in_specs=[
            pl.BlockSpec((blk, h), lambda i: (i, 0)),
            pl.BlockSpec((_NC, blk, h), lambda i: (0, i, 0)),
            pl.BlockSpec((blk, h), lambda i: (i, 0)),
            pl.BlockSpec((h, h), lambda i: (0, 0)),
            pl.BlockSpec((1, h), lambda i: (0, 0)),
            pl.BlockSpec((1, 1), lambda i: (0, 0)),
            pl.BlockSpec((1, 1), lambda i: (0, 0)),
        ],
        out_specs=pl.BlockSpec((blk, h), lambda i: (i, 0)),
        out_shape=jax.ShapeDtypeStruct((n, h), jnp.float32),
    )(emb, parts, dinv, w2, b22, alpha2, beta2)


def _sc_degree(dst, ones_rows, zrows, npad, h, e, k_chunk):
    nw = _NC * _NS
    epw = e // nw
    nchunk = epw // k_chunk
    nps = npad // _NS
    mesh = plsc.VectorSubcoreMesh(core_axis_name="c", subcore_axis_name="s",
                                  num_cores=_NC, num_subcores=_NS)

    @functools.partial(
        pl.kernel,
        out_type=jax.ShapeDtypeStruct((_NC, npad, h), jnp.float32),
        mesh=mesh,
        scratch_types=[
            pltpu.VMEM((k_chunk,), jnp.int32),
            pltpu.VMEM((k_chunk, h), jnp.float32),
            pltpu.VMEM_SHARED((npad, h), jnp.float32),
        ],
    )
    def deg_kernel(dst_h, ones_h, z_h, out_h, di, ones_v, acc):
        c = lax.axis_index("c")
        s = lax.axis_index("s")
        wid = c * _NS + s
        row0 = s * nps
        pltpu.sync_copy(ones_h, ones_v)
        pltpu.sync_copy(z_h, acc.at[pl.ds(row0, nps)])
        plsc.subcore_barrier()
        base = wid * epw

        def body(g, carry):
            off = base + g * k_chunk
            pltpu.sync_copy(dst_h.at[pl.ds(off, k_chunk)], di)
            pltpu.sync_copy(ones_v, acc.at[di], add=True)
            return carry

        lax.fori_loop(0, nchunk, body, 0)
        plsc.subcore_barrier()
        pltpu.sync_copy(acc.at[pl.ds(row0, nps)], out_h.at[c, pl.ds(row0, nps)])

    return deg_kernel(dst, ones_rows, zrows)


def _sc_segsum(emb, src, dst, zrows, npad, h, e, k_chunk):
    nw = _NC * _NS
    epw = e // nw
    nchunk = epw // k_chunk
    nps = npad // _NS
    mesh = plsc.VectorSubcoreMesh(core_axis_name="c", subcore_axis_name="s",
                                  num_cores=_NC, num_subcores=_NS)

    @functools.partial(
        pl.kernel,
        out_type=jax.ShapeDtypeStruct((_NC, npad, h), jnp.float32),
        mesh=mesh,
        scratch_types=[
            pltpu.VMEM((k_chunk,), jnp.int32),
            pltpu.VMEM((k_chunk,), jnp.int32),
            pltpu.VMEM((k_chunk, h), jnp.float32),
            pltpu.VMEM_SHARED((npad, h), jnp.float32),
            pltpu.SemaphoreType.DMA,
        ],
    )
    def seg_kernel(emb_h, src_h, dst_h, z_h, out_h, si, di, rows, acc, sem):
        c = lax.axis_index("c")
        s = lax.axis_index("s")
        wid = c * _NS + s
        row0 = s * nps
        pltpu.sync_copy(z_h, acc.at[pl.ds(row0, nps)])
        plsc.subcore_barrier()
        base = wid * epw

        def body(g, carry):
            off = base + g * k_chunk
            pltpu.sync_copy(src_h.at[pl.ds(off, k_chunk)], si)
            pltpu.sync_copy(dst_h.at[pl.ds(off, k_chunk)], di)
            pltpu.async_copy(emb_h.at[si], rows, sem).wait()
            pltpu.sync_copy(rows, acc.at[di], add=True)
            return carry

        lax.fori_loop(0, nchunk, body, 0)
        plsc.subcore_barrier()
        pltpu.sync_copy(acc.at[pl.ds(row0, nps)], out_h.at[c, pl.ds(row0, nps)])

    return seg_kernel(emb, src, dst, zrows)


def kernel(feature, semantic, edge_index, W1s, b1s, W1m, b1m, W2, b2, alpha,
           beta, num_iterations):
    n, _ = feature.shape
    h = W2.shape[0]
    e = edge_index.shape[1]
    nw = _NC * _NS
    assert e % nw == 0

    npad = -(-n // (_NS * 8)) * (_NS * 8)

    src = edge_index[0]
    dst = edge_index[1]
    k_chunk = _pick_chunk(e // nw)

    alpha2 = jnp.asarray(alpha, jnp.float32).reshape(1, 1)
    beta2 = jnp.asarray(beta, jnp.float32).reshape(1, 1)
    b1s2 = b1s.reshape(1, -1)
    b1m2 = b1m.reshape(1, -1)
    b22 = b2.reshape(1, -1)

    nps = npad // _NS
    zrows = jnp.zeros((nps, h), jnp.float32)
    ones_rows = jnp.ones((k_chunk, h), jnp.float32)

    degp = _sc_degree(dst, ones_rows, zrows, npad, h, e, k_chunk)
    dinv = _tc_dinv(degp, npad, h)
    emb0 = _tc_init(feature, semantic, W1s, b1s2, W1m, b1m2, alpha2, beta2,
                    blk=2000)

    def body(_, emb):
        parts = _sc_segsum(emb, src, dst, zrows, npad, h, e, k_chunk)
        return _tc_update(emb, parts, dinv, W2, b22, alpha2, beta2, blk=2000)

    return lax.fori_loop(0, num_iterations, body, emb0)


# speedup vs baseline: 4.9382x; 1.0004x over previous
import functools

import jax
import jax.numpy as jnp
from jax import lax
from jax.experimental import pallas as pl
from jax.experimental.pallas import tpu as pltpu
from jax.experimental.pallas import tpu_sc as plsc

_NC = 2
_NS = 16
_K = 80
_SLOPE = 0.01


def _lrelu(x):
    return jnp.where(x >= 0, x, _SLOPE * x)


def _tc_init(feature, semantic, w1s, b1s2, w1m, b1m2, alpha2, beta2, blk):
    n, f = feature.shape
    s_dim = semantic.shape[1]
    h = w1s.shape[0]

    def body(f_r, se_r, w1s_r, b1s_r, w1m_r, b1m_r, a_r, b_r, o_r):
        a = a_r[0, 0]
        b = b_r[0, 0]
        x = lax.dot_general(f_r[...], w1s_r[...], (((1,), (1,)), ((), ())),
                            preferred_element_type=jnp.float32) + b1s_r[...]
        y = lax.dot_general(se_r[...], w1m_r[...], (((1,), (1,)), ((), ())),
                            preferred_element_type=jnp.float32) + b1m_r[...]
        o_r[...] = a * _lrelu(x) + b * _lrelu(y)

    return pl.pallas_call(
        body,
        grid=(n // blk,),
        in_specs=[
            pl.BlockSpec((blk, f), lambda i: (i, 0)),
            pl.BlockSpec((blk, s_dim), lambda i: (i, 0)),
            pl.BlockSpec((h, f), lambda i: (0, 0)),
            pl.BlockSpec((1, h), lambda i: (0, 0)),
            pl.BlockSpec((h, s_dim), lambda i: (0, 0)),
            pl.BlockSpec((1, h), lambda i: (0, 0)),
            pl.BlockSpec((1, 1), lambda i: (0, 0)),
            pl.BlockSpec((1, 1), lambda i: (0, 0)),
        ],
        out_specs=pl.BlockSpec((blk, h), lambda i: (i, 0)),
        out_shape=jax.ShapeDtypeStruct((n, h), jnp.float32),
    )(feature, semantic, w1s, b1s2, w1m, b1m2, alpha2, beta2)


def _tc_dinv(degparts, npad, h):
    def body(d_r, o_r):
        dv = d_r[...]
        o_r[...] = 1.0 / jnp.maximum(dv[0] + dv[1], 1.0)

    return pl.pallas_call(
        body,
        out_shape=jax.ShapeDtypeStruct((npad, h), jnp.float32),
    )(degparts)


def _tc_update(emb, parts, dinv, w2, b22, alpha2, beta2, blk):
    n, h = emb.shape

    def body(e_r, p_r, d_r, w2_r, b2_r, a_r, b_r, o_r):
        a = a_r[0, 0]
        b = b_r[0, 0]
        pv = p_r[...]
        msum = pv[0] + pv[1]
        comb = a * e_r[...] + b * (msum * d_r[...])
        z = lax.dot_general(comb, w2_r[...], (((1,), (1,)), ((), ())),
                            preferred_element_type=jnp.float32) + b2_r[...]
        o_r[...] = _lrelu(z)

    return pl.pallas_call(
        body,
        grid=(n // blk,),
        in_specs=[
            pl.BlockSpec((blk, h), lambda i: (i, 0)),
            pl.BlockSpec((_NC, blk, h), lambda i: (0, i, 0)),
            pl.BlockSpec((blk, h), lambda i: (i, 0)),
            pl.BlockSpec((h, h), lambda i: (0, 0)),
            pl.BlockSpec((1, h), lambda i: (0, 0)),
            pl.BlockSpec((1, 1), lambda i: (0, 0)),
            pl.BlockSpec((1, 1), lambda i: (0, 0)),
        ],
        out_specs=pl.BlockSpec((blk, h), lambda i: (i, 0)),
        out_shape=jax.ShapeDtypeStruct((n, h), jnp.float32),
    )(emb, parts, dinv, w2, b22, alpha2, beta2)


def _sc_degree(dst, ones_rows, zrows, npad, h, e):
    nps = npad // _NS
    epw = e // (_NC * _NS)
    nchunk = epw // _K
    mesh = plsc.VectorSubcoreMesh(core_axis_name="c", subcore_axis_name="s",
                                  num_cores=_NC, num_subcores=_NS)

    @functools.partial(
        pl.kernel,
        out_type=jax.ShapeDtypeStruct((_NC, npad, h), jnp.float32),
        mesh=mesh,
        scratch_types=[
            pltpu.VMEM((_K,), jnp.int32),
            pltpu.VMEM((_K, h), jnp.float32),
            pltpu.VMEM_SHARED((npad, h), jnp.float32),
        ],
    )
    def deg_kernel(dst_h, ones_h, z_h, out_h, di, ones_v, acc):
        c = lax.axis_index("c")
        s = lax.axis_index("s")
        wid = c * _NS + s
        row0 = s * nps
        pltpu.sync_copy(ones_h, ones_v)
        pltpu.sync_copy(z_h, acc.at[pl.ds(row0, nps)])
        plsc.subcore_barrier()
        base = wid * epw

        def body(g, carry):
            pltpu.sync_copy(dst_h.at[pl.ds(base + g * _K, _K)], di)
            pltpu.sync_copy(ones_v, acc.at[di], add=True)
            return carry

        lax.fori_loop(0, nchunk, body, 0)
        plsc.subcore_barrier()
        pltpu.sync_copy(acc.at[pl.ds(row0, nps)], out_h.at[c, pl.ds(row0, nps)])

    return deg_kernel(dst, ones_rows, zrows)


def _sc_segsum(emb, src, dst, zrows, npad, h, e):
    nps = npad // _NS
    epw = e // (_NC * _NS)
    nchunk = epw // _K
    mesh = plsc.VectorSubcoreMesh(core_axis_name="c", subcore_axis_name="s",
                                  num_cores=_NC, num_subcores=_NS)

    @functools.partial(
        pl.kernel,
        out_type=jax.ShapeDtypeStruct((_NC, npad, h), jnp.float32),
        mesh=mesh,
        scratch_types=[
            pltpu.VMEM((_K,), jnp.int32),
            pltpu.VMEM((_K,), jnp.int32),
            pltpu.VMEM((_K, h), jnp.float32),
            pltpu.VMEM_SHARED((npad, h), jnp.float32),
            pltpu.SemaphoreType.DMA,
        ],
    )
    def seg_kernel(emb_h, src_h, dst_h, z_h, out_h, si, di, rows, acc, gsem):
        c = lax.axis_index("c")
        s = lax.axis_index("s")
        wid = c * _NS + s
        row0 = s * nps
        pltpu.sync_copy(z_h, acc.at[pl.ds(row0, nps)])
        plsc.subcore_barrier()
        base = wid * epw

        def body(g, carry):
            off = base + g * _K
            pltpu.sync_copy(src_h.at[pl.ds(off, _K)], si)
            pltpu.sync_copy(dst_h.at[pl.ds(off, _K)], di)
            pltpu.async_copy(emb_h.at[si], rows, gsem).wait()
            pltpu.sync_copy(rows, acc.at[di], add=True)
            return carry

        lax.fori_loop(0, nchunk, body, 0)
        plsc.subcore_barrier()
        pltpu.sync_copy(acc.at[pl.ds(row0, nps)], out_h.at[c, pl.ds(row0, nps)])

    return seg_kernel(emb, src, dst, zrows)


def kernel(feature, semantic, edge_index, W1s, b1s, W1m, b1m, W2, b2, alpha,
           beta, num_iterations):
    n, _ = feature.shape
    h = W2.shape[0]
    e = edge_index.shape[1]
    assert e % (_NC * _NS * _K) == 0

    npad = -(-n // (_NS * 8)) * (_NS * 8)

    src = edge_index[0]
    dst = edge_index[1]

    alpha2 = jnp.asarray(alpha, jnp.float32).reshape(1, 1)
    beta2 = jnp.asarray(beta, jnp.float32).reshape(1, 1)
    b1s2 = b1s.reshape(1, -1)
    b1m2 = b1m.reshape(1, -1)
    b22 = b2.reshape(1, -1)

    nps = npad // _NS
    zrows = jnp.zeros((nps, h), jnp.float32)
    ones_rows = jnp.ones((_K, h), jnp.float32)

    degp = _sc_degree(dst, ones_rows, zrows, npad, h, e)
    dinv = _tc_dinv(degp, npad, h)
    emb0 = _tc_init(feature, semantic, W1s, b1s2, W1m, b1m2, alpha2, beta2,
                    blk=2000)

    def body(_, emb):
        parts = _sc_segsum(emb, src, dst, zrows, npad, h, e)
        return _tc_update(emb, parts, dinv, W2, b22, alpha2, beta2, blk=2000)

    return lax.fori_loop(0, num_iterations, body, emb0)
